# baseline (speedup 1.0000x reference)
import functools

import jax
import jax.numpy as jnp
from jax.experimental import pallas as pl
from jax.experimental.pallas import tpu as pltpu

_G = 4
_EPG = 4
_E = _G * _EPG
_TOPK = 2
_GTOPK = 2
_EPS = 1e-9
_FF_T = 2048


def _top2_of4(cols):
    c0, c1, c2, c3 = cols
    m1 = jnp.maximum(jnp.maximum(c0, c1), jnp.maximum(c2, c3))
    i1 = jnp.where(c0 == m1, 0,
         jnp.where(c1 == m1, 1,
         jnp.where(c2 == m1, 2, 3))).astype(jnp.int32)
    neg = jnp.float32(-jnp.inf)
    masked = [jnp.where(i1 == g, neg, cols[g]) for g in range(4)]
    m2 = jnp.maximum(jnp.maximum(masked[0], masked[1]),
                     jnp.maximum(masked[2], masked[3]))
    i2 = jnp.where(masked[0] == m2, 0,
         jnp.where(masked[1] == m2, 1,
         jnp.where(masked[2] == m2, 2, 3))).astype(jnp.int32)
    denom = m1 + m2 + _EPS
    out = []
    for g in range(4):
        w = jnp.where(i1 == g, m1, jnp.where(i2 == g, m2, 0.0)) / denom
        out.append(w)
    return out


def _moe_kernel(x_ref, wr_ref, wg_ref, wu_ref, wd_ref, out_ref, comb_ref):
    e = pl.program_id(0)
    f = pl.program_id(1)

    @pl.when(jnp.logical_and(e == 0, f == 0))
    def _route():
        xf = x_ref[...]
        wr = wr_ref[...]
        scores = jax.nn.sigmoid(
            jax.lax.dot_general(xf, wr, (((1,), (1,)), ((), ())),
                                preferred_element_type=jnp.float32))
        macro_cols = [scores[:, g:g + 1] for g in range(_G)]
        macro_w_cols = _top2_of4(macro_cols)
        for g in range(_G):
            micro_cols = [scores[:, _G + g * _EPG + j:_G + g * _EPG + j + 1]
                          for j in range(_EPG)]
            micro_w_cols = _top2_of4(micro_cols)
            for j in range(_EPG):
                comb_ref[g * _EPG + j] = macro_w_cols[g] * micro_w_cols[j]

    xf = x_ref[...]
    wg = wg_ref[0]
    wu = wu_ref[0]
    wd = wd_ref[0]
    dn = (((1,), (1,)), ((), ()))
    g1 = jax.lax.dot_general(xf, wg, dn, preferred_element_type=jnp.float32)
    up = jax.lax.dot_general(xf, wu, dn, preferred_element_type=jnp.float32)
    h = (g1 * jax.nn.sigmoid(g1)) * up
    h = h * comb_ref[e]
    part = jax.lax.dot_general(h, wd, dn, preferred_element_type=jnp.float32)

    @pl.when(jnp.logical_and(e == 0, f == 0))
    def _init():
        out_ref[...] = part

    @pl.when(jnp.logical_not(jnp.logical_and(e == 0, f == 0)))
    def _acc():
        out_ref[...] += part


def kernel(x, macro_w, micro_w, w_gate, w_up, w_down):
    bsz, seq_len, d_model = x.shape
    T = bsz * seq_len
    E, FF, D = w_gate.shape
    xf = x.reshape(T, d_model)

    wr = jnp.concatenate([macro_w, micro_w.reshape(E, D)], axis=0)
    wr = jnp.pad(wr, ((0, 32 - _G - E), (0, 0)))

    nf = FF // _FF_T
    grid = (E, nf)
    out = pl.pallas_call(
        _moe_kernel,
        grid=grid,
        in_specs=[
            pl.BlockSpec((T, D), lambda e, f: (0, 0)),
            pl.BlockSpec((32, D), lambda e, f: (0, 0)),
            pl.BlockSpec((1, _FF_T, D), lambda e, f: (e, f, 0)),
            pl.BlockSpec((1, _FF_T, D), lambda e, f: (e, f, 0)),
            pl.BlockSpec((1, D, _FF_T), lambda e, f: (e, 0, f)),
        ],
        out_specs=pl.BlockSpec((T, D), lambda e, f: (0, 0)),
        out_shape=jax.ShapeDtypeStruct((T, D), jnp.float32),
        scratch_shapes=[pltpu.VMEM((E, T, 1), jnp.float32)],
        compiler_params=pltpu.CompilerParams(
            dimension_semantics=("arbitrary", "arbitrary"),
            vmem_limit_bytes=100 * 1024 * 1024),
    )(xf, wr, w_gate, w_up, w_down)
    return out.reshape(bsz, seq_len, d_model)


# speedup vs baseline: 1.0064x; 1.0064x over previous
import functools

import jax
import jax.numpy as jnp
from jax.experimental import pallas as pl
from jax.experimental.pallas import tpu as pltpu

_G = 4
_EPG = 4
_E = _G * _EPG
_TOPK = 2
_GTOPK = 2
_EPS = 1e-9
_FF_T = 1024


def _top2_of4(cols):
    c0, c1, c2, c3 = cols
    m1 = jnp.maximum(jnp.maximum(c0, c1), jnp.maximum(c2, c3))
    i1 = jnp.where(c0 == m1, 0,
         jnp.where(c1 == m1, 1,
         jnp.where(c2 == m1, 2, 3))).astype(jnp.int32)
    neg = jnp.float32(-jnp.inf)
    masked = [jnp.where(i1 == g, neg, cols[g]) for g in range(4)]
    m2 = jnp.maximum(jnp.maximum(masked[0], masked[1]),
                     jnp.maximum(masked[2], masked[3]))
    i2 = jnp.where(masked[0] == m2, 0,
         jnp.where(masked[1] == m2, 1,
         jnp.where(masked[2] == m2, 2, 3))).astype(jnp.int32)
    denom = m1 + m2 + _EPS
    out = []
    for g in range(4):
        w = jnp.where(i1 == g, m1, jnp.where(i2 == g, m2, 0.0)) / denom
        out.append(w)
    return out


def _moe_kernel(x_ref, wr_ref, wg_ref, wu_ref, wd_ref, out_ref, comb_ref):
    e = pl.program_id(0)
    f = pl.program_id(1)

    @pl.when(jnp.logical_and(e == 0, f == 0))
    def _route():
        xf = x_ref[...]
        wr = wr_ref[...]
        scores = jax.nn.sigmoid(
            jax.lax.dot_general(xf, wr, (((1,), (1,)), ((), ())),
                                preferred_element_type=jnp.float32))
        macro_cols = [scores[:, g:g + 1] for g in range(_G)]
        macro_w_cols = _top2_of4(macro_cols)
        for g in range(_G):
            micro_cols = [scores[:, _G + g * _EPG + j:_G + g * _EPG + j + 1]
                          for j in range(_EPG)]
            micro_w_cols = _top2_of4(micro_cols)
            for j in range(_EPG):
                comb_ref[g * _EPG + j] = macro_w_cols[g] * micro_w_cols[j]

    xf = x_ref[...]
    wg = wg_ref[0]
    wu = wu_ref[0]
    wd = wd_ref[0]
    dn = (((1,), (1,)), ((), ()))
    g1 = jax.lax.dot_general(xf, wg, dn, preferred_element_type=jnp.float32)
    up = jax.lax.dot_general(xf, wu, dn, preferred_element_type=jnp.float32)
    h = (g1 * jax.nn.sigmoid(g1)) * up
    h = h * comb_ref[e]
    part = jax.lax.dot_general(h, wd, dn, preferred_element_type=jnp.float32)

    @pl.when(jnp.logical_and(e == 0, f == 0))
    def _init():
        out_ref[...] = part

    @pl.when(jnp.logical_not(jnp.logical_and(e == 0, f == 0)))
    def _acc():
        out_ref[...] += part


def kernel(x, macro_w, micro_w, w_gate, w_up, w_down):
    bsz, seq_len, d_model = x.shape
    T = bsz * seq_len
    E, FF, D = w_gate.shape
    xf = x.reshape(T, d_model)

    wr = jnp.concatenate([macro_w, micro_w.reshape(E, D)], axis=0)
    wr = jnp.pad(wr, ((0, 32 - _G - E), (0, 0)))

    nf = FF // _FF_T
    grid = (E, nf)
    out = pl.pallas_call(
        _moe_kernel,
        grid=grid,
        in_specs=[
            pl.BlockSpec((T, D), lambda e, f: (0, 0)),
            pl.BlockSpec((32, D), lambda e, f: (0, 0)),
            pl.BlockSpec((1, _FF_T, D), lambda e, f: (e, f, 0)),
            pl.BlockSpec((1, _FF_T, D), lambda e, f: (e, f, 0)),
            pl.BlockSpec((1, D, _FF_T), lambda e, f: (e, 0, f)),
        ],
        out_specs=pl.BlockSpec((T, D), lambda e, f: (0, 0)),
        out_shape=jax.ShapeDtypeStruct((T, D), jnp.float32),
        scratch_shapes=[pltpu.VMEM((E, T, 1), jnp.float32)],
        compiler_params=pltpu.CompilerParams(
            dimension_semantics=("arbitrary", "arbitrary"),
            vmem_limit_bytes=100 * 1024 * 1024),
    )(xf, wr, w_gate, w_up, w_down)
    return out.reshape(bsz, seq_len, d_model)
